# Initial kernel scaffold; baseline (speedup 1.0000x reference)
import jax
import jax.numpy as jnp
from jax.experimental import pallas as pl


def kernel(x, t_slot, y, y_t_slot, vecs_use, I_array, cand_table, time_embeddings, Ws1, bs1, Ws2, bs2, Wo1, bo1, Wo2, bo2, Wi1, bi1, Wi2, bi2):
    raise NotImplementedError("write your pallas kernel here")


# speedup vs baseline: 11.3625x; 11.3625x over previous
import functools

import numpy as np

import jax
import jax.numpy as jnp
from jax import lax
from jax.experimental import pallas as pl
from jax.experimental.pallas import tpu as pltpu
from jax.experimental.pallas import tpu_sc as plsc

SEQ = 50
USER = 1024
T = SEQ * USER
NCENT = 64
NCAND = 64
D = 20
NW = 32
TPW = T // NW
CPW = (NCENT * NCAND) // NW

_E1 = np.float32(np.e)


def _dotl(a, b):
    return jnp.dot(a.astype(jnp.bfloat16), b.astype(jnp.bfloat16),
                   preferred_element_type=jnp.float32)


PADW = 128
CH = 400
NCHUNK = TPW // CH


def _sc_gather(vpad, xv, cand_flat):
    mesh = plsc.VectorSubcoreMesh(core_axis_name="c", subcore_axis_name="s")

    @functools.partial(
        pl.kernel,
        out_type=[
            jax.ShapeDtypeStruct((T, PADW), jnp.float32),
            jax.ShapeDtypeStruct((NCENT * NCAND, PADW), jnp.float32),
        ],
        mesh=mesh,
        scratch_types=[
            [pltpu.VMEM((CH,), jnp.int32) for _ in range(NCHUNK)],
            pltpu.VMEM((CH, PADW), jnp.float32),
            pltpu.VMEM((CPW,), jnp.int32),
            pltpu.VMEM((CPW, PADW), jnp.float32),
            pltpu.SemaphoreType.DMA,
        ],
    )
    def k(vpad_hbm, xv_hbm, cf_hbm, xemb_out, candr_out,
          idx_vs, rows_v, cidx_v, crow_v, sem):
        wid = lax.axis_index("s") * 2 + lax.axis_index("c")
        base = wid * TPW
        for c in range(NCHUNK):
            pltpu.sync_copy(xv_hbm.at[pl.ds(base + c * CH, CH)], idx_vs[c])
            pltpu.async_copy(vpad_hbm.at[idx_vs[c]], rows_v, sem).wait()
            pltpu.sync_copy(rows_v, xemb_out.at[pl.ds(base + c * CH, CH)])
        cbase = wid * CPW
        pltpu.sync_copy(cf_hbm.at[pl.ds(cbase, CPW)], cidx_v)
        pltpu.async_copy(vpad_hbm.at[cidx_v], crow_v, sem).wait()
        pltpu.sync_copy(crow_v, candr_out.at[pl.ds(cbase, CPW)])

    return k(vpad, xv, cand_flat)


def _prep_body(candr_ref, te2_ref, wi1a_ref, wi1b_ref, bi1_ref, wi2_ref,
               bi2_ref, out_ref):
    cr = candr_ref[...]
    h = (_dotl(cr, wi1a_ref[...]) + _dotl(te2_ref[...], wi1b_ref[...])
         + bi1_ref[...])
    h = jnp.maximum(h, 0.0)
    out_ref[...] = _dotl(h, wi2_ref[...]) + bi2_ref[...]


def _prep(candr, te2, wi1a, wi1b, bi1, wi2, bi2):
    return pl.pallas_call(
        _prep_body,
        out_shape=jax.ShapeDtypeStruct((NCENT * NCAND, D), jnp.float32),
    )(candr, te2, wi1a, wi1b, bi1, wi2, bi2)


def _main_body(x4_ref, x3_ref, x2_ref, x1_ref, x0_ref, tsl_ref, cent_ref,
               tcat_ref, rcat_ref, kmat_ref, smat_ref,
               w1_4_ref, w1_3_ref, w1_2_ref, w1_1_ref, w1_0_ref, bs1_ref,
               ws2_ref, bs2_ref, woa_ref, wob_ref, bo1_ref, wo2_ref, bo2_ref,
               te_ref, out_ref):
    f32 = jnp.float32
    dot = functools.partial(jnp.dot, preferred_element_type=f32,
                            precision=jax.lax.Precision.HIGHEST)
    xcur = x0_ref[...]

    h1 = (_dotl(x4_ref[...], w1_4_ref[...]) + _dotl(x3_ref[...], w1_3_ref[...])
          + _dotl(x2_ref[...], w1_2_ref[...]) + _dotl(x1_ref[...], w1_1_ref[...])
          + _dotl(xcur, w1_0_ref[...]) + bs1_ref[...])
    h1 = jnp.maximum(h1, 0.0)
    hist_e = _dotl(h1, ws2_ref[...]) + bs2_ref[...]

    tv = tsl_ref[...]
    hh = tv % 24
    seg = jnp.where((hh >= 22) | (hh < 6), 0,
                    jnp.where((hh >= 6) & (hh < 14), 1, 2))
    seg4 = lax.broadcasted_iota(jnp.int32, (USER, 4), 1)
    ht = (seg4 == seg).astype(f32)
    xte = dot(ht, te_ref[...])
    xh = (_dotl(xcur, woa_ref[...]) + _dotl(xte, wob_ref[...])
          + bo1_ref[...])
    xh = jnp.maximum(xh, 0.0)
    xi = _dotl(xh, wo2_ref[...]) + bo2_ref[...]

    q = (hist_e + xi) * 0.5

    cent = cent_ref[...]
    lane64 = lax.broadcasted_iota(jnp.int32, (USER, NCAND), 1)
    hsel = (lane64 == cent).astype(f32)
    ce = dot(hsel, tcat_ref[...])
    qb = dot(q, kmat_ref[...])
    df = ce - qb
    sq = df * df
    dist2 = sq[:, 0:NCAND]
    for d in range(1, D):
        dist2 = dist2 + sq[:, d * NCAND:(d + 1) * NCAND]
    dist = jnp.sqrt(dist2 + 1e-12)
    score = jnp.exp(-0.02 * dist)

    rem = score
    mask = jnp.zeros((USER, NCAND), dtype=jnp.bool_)
    for _ in range(10):
        mx = jnp.max(rem, axis=1, keepdims=True)
        eq = rem == mx
        first = jnp.min(jnp.where(eq, lane64, NCAND), axis=1, keepdims=True)
        pick = lane64 == first
        mask = mask | pick
        rem = jnp.where(pick, -1.0, rem)

    w = jnp.where(mask, jnp.exp(score), 0.0)
    den = jnp.sum(w, axis=1, keepdims=True) + _E1

    cer = dot(hsel, rcat_ref[...])
    wrep = jnp.concatenate([w] * D, axis=1)
    num = dot(cer * wrep, smat_ref[...])
    out_ref[...] = (num + _E1 * xcur) / den


def _main(x_emb, tsl_col, cent_col, tcat, rcat, kmat, smat,
          w1s, bs1, ws2, bs2, woa, wob, bo1, wo2, bo2, te):
    def xspec(k):
        return pl.BlockSpec(
            (USER, D),
            lambda s, k=k: (jnp.where(s >= k, s - k, s), 0))

    full = lambda shp: pl.BlockSpec(shp, lambda s: (0,) * len(shp))
    in_specs = [
        xspec(4), xspec(3), xspec(2), xspec(1), xspec(0),
        pl.BlockSpec((USER, 1), lambda s: (s, 0)),
        pl.BlockSpec((USER, 1), lambda s: (s, 0)),
        full((NCENT, NCAND * D)),
        full((NCENT, NCAND * D)),
        full((D, NCAND * D)),
        full((NCAND * D, D)),
        full((D, 40)), full((D, 40)), full((D, 40)), full((D, 40)),
        full((D, 40)),
        full((1, 40)),
        full((40, D)), full((1, D)),
        full((D, 40)), full((D, 40)), full((1, 40)),
        full((40, D)), full((1, D)),
        full((4, D)),
    ]
    return pl.pallas_call(
        _main_body,
        grid=(SEQ,),
        in_specs=in_specs,
        out_specs=pl.BlockSpec((USER, D), lambda s: (s, 0)),
        out_shape=jax.ShapeDtypeStruct((T, D), jnp.float32),
    )(x_emb, x_emb, x_emb, x_emb, x_emb, tsl_col, cent_col,
      tcat, rcat, kmat, smat,
      w1s[0], w1s[1], w1s[2], w1s[3], w1s[4], bs1, ws2, bs2,
      woa, wob, bo1, wo2, bo2, te)


def kernel(x, t_slot, y, y_t_slot, vecs_use, I_array, cand_table,
           time_embeddings, Ws1, bs1, Ws2, bs2, Wo1, bo1, Wo2, bo2,
           Wi1, bi1, Wi2, bi2):
    del y, y_t_slot
    xv = x.reshape(-1).astype(jnp.int32)
    tsl_col = t_slot.reshape(-1, 1).astype(jnp.int32)
    cand_flat = cand_table.reshape(-1).astype(jnp.int32)

    vpad = jnp.pad(
        jnp.concatenate(
            [vecs_use, I_array.astype(jnp.float32)[:, None]], axis=1),
        ((0, 0), (0, PADW - D - 1)))

    x_emb_pad, candr_pad = _sc_gather(vpad, xv, cand_flat)
    x_emb = x_emb_pad[:, :D]
    cent_col = x_emb_pad[:, D:D + 1].astype(jnp.int32)
    candr = candr_pad[:, :D]

    te2 = time_embeddings[2:3]
    tabt = _prep(candr, te2, Wi1[:D], Wi1[D:], bi1.reshape(1, -1),
                 Wi2, bi2.reshape(1, -1))

    tcat = tabt.reshape(NCENT, NCAND, D).transpose(0, 2, 1).reshape(
        NCENT, D * NCAND)
    rcat = candr.reshape(NCENT, NCAND, D).transpose(0, 2, 1).reshape(
        NCENT, D * NCAND)

    kmat = jnp.asarray(np.kron(np.eye(D, dtype=np.float32),
                               np.ones((1, NCAND), np.float32)))
    smat = jnp.asarray(np.kron(np.eye(D, dtype=np.float32),
                               np.ones((NCAND, 1), np.float32)))

    w1s = [Ws1[0:D], Ws1[D:2 * D], Ws1[2 * D:3 * D], Ws1[3 * D:4 * D],
           Ws1[4 * D:5 * D]]
    return _main(x_emb, tsl_col, cent_col, tcat, rcat, kmat, smat,
                 w1s, bs1.reshape(1, -1), Ws2, bs2.reshape(1, -1),
                 Wo1[:D], Wo1[D:], bo1.reshape(1, -1), Wo2,
                 bo2.reshape(1, -1), time_embeddings)


# speedup vs baseline: 15.6358x; 1.3761x over previous
import functools

import numpy as np

import jax
import jax.numpy as jnp
from jax import lax
from jax.experimental import pallas as pl
from jax.experimental.pallas import tpu as pltpu
from jax.experimental.pallas import tpu_sc as plsc

SEQ = 50
USER = 1024
T = SEQ * USER
NCENT = 64
NCAND = 64
D = 20
NW = 32
TPW = T // NW
CPW = (NCENT * NCAND) // NW

_E1 = np.float32(np.e)


def _dotl(a, b):
    return jnp.dot(a.astype(jnp.bfloat16), b.astype(jnp.bfloat16),
                   preferred_element_type=jnp.float32)


PADW = 128
CH = 400
NCHUNK = TPW // CH


def _sc_gather(vpad, xv, cand_flat):
    mesh = plsc.VectorSubcoreMesh(core_axis_name="c", subcore_axis_name="s")

    @functools.partial(
        pl.kernel,
        out_type=[
            jax.ShapeDtypeStruct((T, PADW), jnp.float32),
            jax.ShapeDtypeStruct((NCENT * NCAND, PADW), jnp.float32),
        ],
        mesh=mesh,
        scratch_types=[
            [pltpu.VMEM((CH,), jnp.int32) for _ in range(NCHUNK)],
            pltpu.VMEM((CH, PADW), jnp.float32),
            pltpu.VMEM((CPW,), jnp.int32),
            pltpu.VMEM((CPW, PADW), jnp.float32),
            pltpu.SemaphoreType.DMA,
        ],
    )
    def k(vpad_hbm, xv_hbm, cf_hbm, xemb_out, candr_out,
          idx_vs, rows_v, cidx_v, crow_v, sem):
        wid = lax.axis_index("s") * 2 + lax.axis_index("c")
        base = wid * TPW
        for c in range(NCHUNK):
            pltpu.sync_copy(xv_hbm.at[pl.ds(base + c * CH, CH)], idx_vs[c])
            pltpu.async_copy(vpad_hbm.at[idx_vs[c]], rows_v, sem).wait()
            pltpu.sync_copy(rows_v, xemb_out.at[pl.ds(base + c * CH, CH)])
        cbase = wid * CPW
        pltpu.sync_copy(cf_hbm.at[pl.ds(cbase, CPW)], cidx_v)
        pltpu.async_copy(vpad_hbm.at[cidx_v], crow_v, sem).wait()
        pltpu.sync_copy(crow_v, candr_out.at[pl.ds(cbase, CPW)])

    return k(vpad, xv, cand_flat)


def _prep_body(candr_ref, te2_ref, wi1a_ref, wi1b_ref, bi1_ref, wi2_ref,
               bi2_ref, out_ref):
    cr = candr_ref[...]
    h = (_dotl(cr, wi1a_ref[...]) + _dotl(te2_ref[...], wi1b_ref[...])
         + bi1_ref[...])
    h = jnp.maximum(h, 0.0)
    out_ref[...] = _dotl(h, wi2_ref[...]) + bi2_ref[...]


def _prep(candr, te2, wi1a, wi1b, bi1, wi2, bi2):
    return pl.pallas_call(
        _prep_body,
        out_shape=jax.ShapeDtypeStruct((NCENT * NCAND, D), jnp.float32),
    )(candr, te2, wi1a, wi1b, bi1, wi2, bi2)


def _main_body(x4_ref, x3_ref, x2_ref, x1_ref, x0_ref, tsl_ref,
               tcat_ref, rcat_ref,
               w1_4_ref, w1_3_ref, w1_2_ref, w1_1_ref, w1_0_ref, bs1_ref,
               ws2_ref, bs2_ref, woa_ref, wob_ref, bo1_ref, wo2_ref, bo2_ref,
               te_ref, out_ref):
    f32 = jnp.float32
    dot = functools.partial(jnp.dot, preferred_element_type=f32,
                            precision=jax.lax.Precision.HIGHEST)
    x0 = x0_ref[...]
    xcur = x0[:, :D]
    cent = x0[:, D:D + 1].astype(jnp.int32)

    h1 = (_dotl(x4_ref[:, :D], w1_4_ref[...])
          + _dotl(x3_ref[:, :D], w1_3_ref[...])
          + _dotl(x2_ref[:, :D], w1_2_ref[...])
          + _dotl(x1_ref[:, :D], w1_1_ref[...])
          + _dotl(xcur, w1_0_ref[...]) + bs1_ref[...])
    h1 = jnp.maximum(h1, 0.0)
    hist_e = _dotl(h1, ws2_ref[...]) + bs2_ref[...]

    tv = tsl_ref[...]
    hh = tv % 24
    seg = jnp.where((hh >= 22) | (hh < 6), 0,
                    jnp.where((hh >= 6) & (hh < 14), 1, 2))
    seg4 = lax.broadcasted_iota(jnp.int32, (USER, 4), 1)
    ht = (seg4 == seg).astype(f32)
    xte = dot(ht, te_ref[...])
    xh = (_dotl(xcur, woa_ref[...]) + _dotl(xte, wob_ref[...])
          + bo1_ref[...])
    xh = jnp.maximum(xh, 0.0)
    xi = _dotl(xh, wo2_ref[...]) + bo2_ref[...]

    q = (hist_e + xi) * 0.5

    lane64 = lax.broadcasted_iota(jnp.int32, (USER, NCAND), 1)
    hsel = (lane64 == cent).astype(f32)
    dist2 = None
    for d in range(D):
        ce_d = dot(hsel, tcat_ref[d * NCAND:(d + 1) * NCAND, :])
        df = ce_d - q[:, d:d + 1]
        dist2 = df * df if dist2 is None else dist2 + df * df
    dist = jnp.sqrt(dist2 + 1e-12)
    score = jnp.exp(-0.02 * dist)

    rem = score
    mask = jnp.zeros((USER, NCAND), dtype=jnp.bool_)
    for _ in range(10):
        mx = jnp.max(rem, axis=1, keepdims=True)
        eq = rem == mx
        first = jnp.min(jnp.where(eq, lane64, NCAND), axis=1, keepdims=True)
        pick = lane64 == first
        mask = mask | pick
        rem = jnp.where(pick, -1.0, rem)

    w = jnp.where(mask, jnp.exp(score), 0.0)
    den = jnp.sum(w, axis=1, keepdims=True) + _E1

    for d in range(D):
        cer_d = dot(hsel, rcat_ref[d * NCAND:(d + 1) * NCAND, :])
        num_d = jnp.sum(w * cer_d, axis=1, keepdims=True)
        out_ref[:, d:d + 1] = (num_d + _E1 * xcur[:, d:d + 1]) / den


def _main(x_emb_pad, tsl_col, tcat, rcat,
          w1s, bs1, ws2, bs2, woa, wob, bo1, wo2, bo2, te):
    def xspec(k):
        return pl.BlockSpec(
            (USER, PADW),
            lambda s, k=k: (jnp.where(s >= k, s - k, s), 0))

    full = lambda shp: pl.BlockSpec(shp, lambda s: (0,) * len(shp))
    in_specs = [
        xspec(4), xspec(3), xspec(2), xspec(1), xspec(0),
        pl.BlockSpec((USER, 1), lambda s: (s, 0)),
        full((D * NCAND, NCAND)),
        full((D * NCAND, NCAND)),
        full((D, 40)), full((D, 40)), full((D, 40)), full((D, 40)),
        full((D, 40)),
        full((1, 40)),
        full((40, D)), full((1, D)),
        full((D, 40)), full((D, 40)), full((1, 40)),
        full((40, D)), full((1, D)),
        full((4, D)),
    ]
    return pl.pallas_call(
        _main_body,
        grid=(SEQ,),
        in_specs=in_specs,
        out_specs=pl.BlockSpec((USER, D), lambda s: (s, 0)),
        out_shape=jax.ShapeDtypeStruct((T, D), jnp.float32),
    )(x_emb_pad, x_emb_pad, x_emb_pad, x_emb_pad, x_emb_pad, tsl_col,
      tcat, rcat,
      w1s[0], w1s[1], w1s[2], w1s[3], w1s[4], bs1, ws2, bs2,
      woa, wob, bo1, wo2, bo2, te)


def kernel(x, t_slot, y, y_t_slot, vecs_use, I_array, cand_table,
           time_embeddings, Ws1, bs1, Ws2, bs2, Wo1, bo1, Wo2, bo2,
           Wi1, bi1, Wi2, bi2):
    del y, y_t_slot
    xv = x.reshape(-1).astype(jnp.int32)
    tsl_col = t_slot.reshape(-1, 1).astype(jnp.int32)
    cand_flat = cand_table.reshape(-1).astype(jnp.int32)

    vpad = jnp.pad(
        jnp.concatenate(
            [vecs_use, I_array.astype(jnp.float32)[:, None]], axis=1),
        ((0, 0), (0, PADW - D - 1)))

    x_emb_pad, candr_pad = _sc_gather(vpad, xv, cand_flat)
    candr = candr_pad[:, :D]

    te2 = time_embeddings[2:3]
    tabt = _prep(candr, te2, Wi1[:D], Wi1[D:], bi1.reshape(1, -1),
                 Wi2, bi2.reshape(1, -1))

    tcat = tabt.reshape(NCENT, NCAND, D).transpose(2, 0, 1).reshape(
        D * NCENT, NCAND)
    rcat = candr.reshape(NCENT, NCAND, D).transpose(2, 0, 1).reshape(
        D * NCENT, NCAND)

    w1s = [Ws1[0:D], Ws1[D:2 * D], Ws1[2 * D:3 * D], Ws1[3 * D:4 * D],
           Ws1[4 * D:5 * D]]
    return _main(x_emb_pad, tsl_col, tcat, rcat,
                 w1s, bs1.reshape(1, -1), Ws2, bs2.reshape(1, -1),
                 Wo1[:D], Wo1[D:], bo1.reshape(1, -1), Wo2,
                 bo2.reshape(1, -1), time_embeddings)


# speedup vs baseline: 16.2684x; 1.0405x over previous
import functools

import numpy as np

import jax
import jax.numpy as jnp
from jax import lax
from jax.experimental import pallas as pl
from jax.experimental.pallas import tpu as pltpu
from jax.experimental.pallas import tpu_sc as plsc

SEQ = 50
USER = 1024
T = SEQ * USER
NCENT = 64
NCAND = 64
D = 20
NW = 32
TPW = T // NW
CPW = (NCENT * NCAND) // NW

_E1 = np.float32(np.e)


def _dotl(a, b):
    return jnp.dot(a.astype(jnp.bfloat16), b.astype(jnp.bfloat16),
                   preferred_element_type=jnp.float32)


PADW = 128
CH = 400
NCHUNK = TPW // CH


def _sc_gather(vpad, xv, cand_flat):
    mesh = plsc.VectorSubcoreMesh(core_axis_name="c", subcore_axis_name="s")

    @functools.partial(
        pl.kernel,
        out_type=[
            jax.ShapeDtypeStruct((T, PADW), jnp.float32),
            jax.ShapeDtypeStruct((NCENT * NCAND, PADW), jnp.float32),
        ],
        mesh=mesh,
        scratch_types=[
            [pltpu.VMEM((CH,), jnp.int32) for _ in range(NCHUNK)],
            pltpu.VMEM((CH, PADW), jnp.float32),
            pltpu.VMEM((CPW,), jnp.int32),
            pltpu.VMEM((CPW, PADW), jnp.float32),
            pltpu.SemaphoreType.DMA,
        ],
    )
    def k(vpad_hbm, xv_hbm, cf_hbm, xemb_out, candr_out,
          idx_vs, rows_v, cidx_v, crow_v, sem):
        wid = lax.axis_index("s") * 2 + lax.axis_index("c")
        base = wid * TPW
        for c in range(NCHUNK):
            pltpu.sync_copy(xv_hbm.at[pl.ds(base + c * CH, CH)], idx_vs[c])
            pltpu.async_copy(vpad_hbm.at[idx_vs[c]], rows_v, sem).wait()
            pltpu.sync_copy(rows_v, xemb_out.at[pl.ds(base + c * CH, CH)])
        cbase = wid * CPW
        pltpu.sync_copy(cf_hbm.at[pl.ds(cbase, CPW)], cidx_v)
        pltpu.async_copy(vpad_hbm.at[cidx_v], crow_v, sem).wait()
        pltpu.sync_copy(crow_v, candr_out.at[pl.ds(cbase, CPW)])

    return k(vpad, xv, cand_flat)


def _prep_body(candr_ref, te2_ref, wi1a_ref, wi1b_ref, bi1_ref, wi2_ref,
               bi2_ref, out_ref):
    cr = candr_ref[...]
    h = (_dotl(cr, wi1a_ref[...]) + _dotl(te2_ref[...], wi1b_ref[...])
         + bi1_ref[...])
    h = jnp.maximum(h, 0.0)
    out_ref[...] = _dotl(h, wi2_ref[...]) + bi2_ref[...]


def _prep(candr, te2, wi1a, wi1b, bi1, wi2, bi2):
    return pl.pallas_call(
        _prep_body,
        out_shape=jax.ShapeDtypeStruct((NCENT * NCAND, D), jnp.float32),
    )(candr, te2, wi1a, wi1b, bi1, wi2, bi2)


def _main_body(x4_ref, x3_ref, x2_ref, x1_ref, x0_ref, tsl_ref,
               tcat_ref, rcat_ref,
               w1_4_ref, w1_3_ref, w1_2_ref, w1_1_ref, w1_0_ref, bs1_ref,
               ws2_ref, bs2_ref, woa_ref, wob_ref, bo1_ref, wo2_ref, bo2_ref,
               te_ref, out_ref):
    f32 = jnp.float32
    dot = functools.partial(jnp.dot, preferred_element_type=f32,
                            precision=jax.lax.Precision.HIGHEST)
    x0 = x0_ref[...]
    xcur = x0[:, :D]
    cent = x0[:, D:D + 1].astype(jnp.int32)

    h1 = (_dotl(x4_ref[:, :D], w1_4_ref[...])
          + _dotl(x3_ref[:, :D], w1_3_ref[...])
          + _dotl(x2_ref[:, :D], w1_2_ref[...])
          + _dotl(x1_ref[:, :D], w1_1_ref[...])
          + _dotl(xcur, w1_0_ref[...]) + bs1_ref[...])
    h1 = jnp.maximum(h1, 0.0)
    hist_e = _dotl(h1, ws2_ref[...]) + bs2_ref[...]

    tv = tsl_ref[...]
    hh = tv % 24
    seg = jnp.where((hh >= 22) | (hh < 6), 0,
                    jnp.where((hh >= 6) & (hh < 14), 1, 2))
    seg4 = lax.broadcasted_iota(jnp.int32, (USER, 4), 1)
    ht = (seg4 == seg).astype(f32)
    xte = dot(ht, te_ref[...])
    xh = (_dotl(xcur, woa_ref[...]) + _dotl(xte, wob_ref[...])
          + bo1_ref[...])
    xh = jnp.maximum(xh, 0.0)
    xi = _dotl(xh, wo2_ref[...]) + bo2_ref[...]

    q = (hist_e + xi) * 0.5

    lane64 = lax.broadcasted_iota(jnp.int32, (USER, NCAND), 1)
    hsel = (lane64 == cent).astype(f32)
    dist2 = None
    for c in range(D // 4):
        ce4 = dot(hsel, tcat_ref[:, c * 256:(c + 1) * 256])
        for dd in range(4):
            d = 4 * c + dd
            df = ce4[:, dd * NCAND:(dd + 1) * NCAND] - q[:, d:d + 1]
            dist2 = df * df if dist2 is None else dist2 + df * df
    dist = jnp.sqrt(dist2 + 1e-12)
    score = jnp.exp(-0.02 * dist)

    rem = score
    mask = jnp.zeros((USER, NCAND), dtype=jnp.bool_)
    for _ in range(10):
        mx = jnp.max(rem, axis=1, keepdims=True)
        eq = rem == mx
        first = jnp.min(jnp.where(eq, lane64, NCAND), axis=1, keepdims=True)
        pick = lane64 == first
        mask = mask | pick
        rem = jnp.where(pick, -1.0, rem)

    w = jnp.where(mask, jnp.exp(score), 0.0)
    den = jnp.sum(w, axis=1, keepdims=True) + _E1

    for c in range(D // 4):
        cer4 = dot(hsel, rcat_ref[:, c * 256:(c + 1) * 256])
        for dd in range(4):
            d = 4 * c + dd
            num_d = jnp.sum(w * cer4[:, dd * NCAND:(dd + 1) * NCAND],
                            axis=1, keepdims=True)
            out_ref[:, d:d + 1] = (num_d + _E1 * xcur[:, d:d + 1]) / den


def _main(x_emb_pad, tsl_col, tcat, rcat,
          w1s, bs1, ws2, bs2, woa, wob, bo1, wo2, bo2, te):
    def xspec(k):
        return pl.BlockSpec(
            (USER, PADW),
            lambda s, k=k: (jnp.where(s >= k, s - k, s), 0))

    full = lambda shp: pl.BlockSpec(shp, lambda s: (0,) * len(shp))
    in_specs = [
        xspec(4), xspec(3), xspec(2), xspec(1), xspec(0),
        pl.BlockSpec((USER, 1), lambda s: (s, 0)),
        full((NCENT, D * NCAND)),
        full((NCENT, D * NCAND)),
        full((D, 40)), full((D, 40)), full((D, 40)), full((D, 40)),
        full((D, 40)),
        full((1, 40)),
        full((40, D)), full((1, D)),
        full((D, 40)), full((D, 40)), full((1, 40)),
        full((40, D)), full((1, D)),
        full((4, D)),
    ]
    return pl.pallas_call(
        _main_body,
        grid=(SEQ,),
        in_specs=in_specs,
        out_specs=pl.BlockSpec((USER, D), lambda s: (s, 0)),
        out_shape=jax.ShapeDtypeStruct((T, D), jnp.float32),
    )(x_emb_pad, x_emb_pad, x_emb_pad, x_emb_pad, x_emb_pad, tsl_col,
      tcat, rcat,
      w1s[0], w1s[1], w1s[2], w1s[3], w1s[4], bs1, ws2, bs2,
      woa, wob, bo1, wo2, bo2, te)


def kernel(x, t_slot, y, y_t_slot, vecs_use, I_array, cand_table,
           time_embeddings, Ws1, bs1, Ws2, bs2, Wo1, bo1, Wo2, bo2,
           Wi1, bi1, Wi2, bi2):
    del y, y_t_slot
    xv = x.reshape(-1).astype(jnp.int32)
    tsl_col = t_slot.reshape(-1, 1).astype(jnp.int32)
    cand_flat = cand_table.reshape(-1).astype(jnp.int32)

    vpad = jnp.pad(
        jnp.concatenate(
            [vecs_use, I_array.astype(jnp.float32)[:, None]], axis=1),
        ((0, 0), (0, PADW - D - 1)))

    x_emb_pad, candr_pad = _sc_gather(vpad, xv, cand_flat)
    candr = candr_pad[:, :D]

    te2 = time_embeddings[2:3]
    tabt = _prep(candr, te2, Wi1[:D], Wi1[D:], bi1.reshape(1, -1),
                 Wi2, bi2.reshape(1, -1))

    tcat = tabt.reshape(NCENT, NCAND, D).transpose(0, 2, 1).reshape(
        NCENT, D * NCAND)
    rcat = candr.reshape(NCENT, NCAND, D).transpose(0, 2, 1).reshape(
        NCENT, D * NCAND)

    w1s = [Ws1[0:D], Ws1[D:2 * D], Ws1[2 * D:3 * D], Ws1[3 * D:4 * D],
           Ws1[4 * D:5 * D]]
    return _main(x_emb_pad, tsl_col, tcat, rcat,
                 w1s, bs1.reshape(1, -1), Ws2, bs2.reshape(1, -1),
                 Wo1[:D], Wo1[D:], bo1.reshape(1, -1), Wo2,
                 bo2.reshape(1, -1), time_embeddings)


# speedup vs baseline: 27.0090x; 1.6602x over previous
import functools

import numpy as np

import jax
import jax.numpy as jnp
from jax import lax
from jax.experimental import pallas as pl
from jax.experimental.pallas import tpu as pltpu
from jax.experimental.pallas import tpu_sc as plsc

SEQ = 50
USER = 1024
T = SEQ * USER
NCENT = 64
NCAND = 64
D = 20
NW = 32
TPW = T // NW
CPW = (NCENT * NCAND) // NW

_E1 = np.float32(np.e)


def _dotl(a, b):
    return jnp.dot(a.astype(jnp.bfloat16), b.astype(jnp.bfloat16),
                   preferred_element_type=jnp.float32)


PADW = 128
CH = 400
NCHUNK = TPW // CH


def _sc_gather(vpad, xv, cand_flat):
    mesh = plsc.VectorSubcoreMesh(core_axis_name="c", subcore_axis_name="s")

    @functools.partial(
        pl.kernel,
        out_type=[
            jax.ShapeDtypeStruct((T, PADW), jnp.float32),
            jax.ShapeDtypeStruct((NCENT * NCAND, PADW), jnp.float32),
        ],
        mesh=mesh,
        scratch_types=[
            [pltpu.VMEM((CH,), jnp.int32) for _ in range(NCHUNK)],
            pltpu.VMEM((CH, PADW), jnp.float32),
            pltpu.VMEM((CPW,), jnp.int32),
            pltpu.VMEM((CPW, PADW), jnp.float32),
            pltpu.SemaphoreType.DMA,
        ],
    )
    def k(vpad_hbm, xv_hbm, cf_hbm, xemb_out, candr_out,
          idx_vs, rows_v, cidx_v, crow_v, sem):
        wid = lax.axis_index("s") * 2 + lax.axis_index("c")
        base = wid * TPW
        for c in range(NCHUNK):
            pltpu.sync_copy(xv_hbm.at[pl.ds(base + c * CH, CH)], idx_vs[c])
            pltpu.async_copy(vpad_hbm.at[idx_vs[c]], rows_v, sem).wait()
            pltpu.sync_copy(rows_v, xemb_out.at[pl.ds(base + c * CH, CH)])
        cbase = wid * CPW
        pltpu.sync_copy(cf_hbm.at[pl.ds(cbase, CPW)], cidx_v)
        pltpu.async_copy(vpad_hbm.at[cidx_v], crow_v, sem).wait()
        pltpu.sync_copy(crow_v, candr_out.at[pl.ds(cbase, CPW)])

    return k(vpad, xv, cand_flat)


def _prep_body(candr_ref, te2_ref, wi1a_ref, wi1b_ref, bi1_ref, wi2_ref,
               bi2_ref, out_ref):
    cr = candr_ref[...]
    h = (_dotl(cr, wi1a_ref[...]) + _dotl(te2_ref[...], wi1b_ref[...])
         + bi1_ref[...])
    h = jnp.maximum(h, 0.0)
    out_ref[...] = _dotl(h, wi2_ref[...]) + bi2_ref[...]


def _prep(candr, te2, wi1a, wi1b, bi1, wi2, bi2):
    return pl.pallas_call(
        _prep_body,
        out_shape=jax.ShapeDtypeStruct((NCENT * NCAND, D), jnp.float32),
    )(candr, te2, wi1a, wi1b, bi1, wi2, bi2)


def _main_body(x4_ref, x3_ref, x2_ref, x1_ref, x0_ref, tsl_ref,
               tcat_ref, rcat_ref,
               w1_4_ref, w1_3_ref, w1_2_ref, w1_1_ref, w1_0_ref, bs1_ref,
               ws2_ref, bs2_ref, woa_ref, wob_ref, bo1_ref, wo2_ref, bo2_ref,
               te_ref, out_ref):
    f32 = jnp.float32
    dot = functools.partial(jnp.dot, preferred_element_type=f32,
                            precision=jax.lax.Precision.HIGHEST)
    x0 = x0_ref[...]
    xcur = x0[:, :D]
    cent = x0[:, D:D + 1].astype(jnp.int32)

    h1 = (_dotl(x4_ref[:, :D], w1_4_ref[...])
          + _dotl(x3_ref[:, :D], w1_3_ref[...])
          + _dotl(x2_ref[:, :D], w1_2_ref[...])
          + _dotl(x1_ref[:, :D], w1_1_ref[...])
          + _dotl(xcur, w1_0_ref[...]) + bs1_ref[...])
    h1 = jnp.maximum(h1, 0.0)
    hist_e = _dotl(h1, ws2_ref[...]) + bs2_ref[...]

    tv = tsl_ref[...]
    hh = tv % 24
    seg = jnp.where((hh >= 22) | (hh < 6), 0,
                    jnp.where((hh >= 6) & (hh < 14), 1, 2))
    seg4 = lax.broadcasted_iota(jnp.int32, (USER, 4), 1)
    ht = (seg4 == seg).astype(f32)
    xte = dot(ht, te_ref[...])
    xh = (_dotl(xcur, woa_ref[...]) + _dotl(xte, wob_ref[...])
          + bo1_ref[...])
    xh = jnp.maximum(xh, 0.0)
    xi = _dotl(xh, wo2_ref[...]) + bo2_ref[...]

    q = (hist_e + xi) * 0.5

    qt = q.T
    xct = xcur.T
    centt = cent.reshape(1, USER)
    sub64 = lax.broadcasted_iota(jnp.int32, (NCAND, USER), 0)
    hselt = (sub64 == centt).astype(f32)
    dist2 = None
    for c in range(D // 4):
        ce4 = dot(tcat_ref[c * 256:(c + 1) * 256, :], hselt)
        for dd in range(4):
            d = 4 * c + dd
            df = ce4[dd * NCAND:(dd + 1) * NCAND, :] - qt[d:d + 1, :]
            dist2 = df * df if dist2 is None else dist2 + df * df
    dist = jnp.sqrt(dist2 + 1e-12)
    score = jnp.exp(-0.02 * dist)

    rem = score
    mask = jnp.zeros((NCAND, USER), dtype=jnp.bool_)
    for _ in range(10):
        mx = jnp.max(rem, axis=0, keepdims=True)
        eq = rem == mx
        first = jnp.min(jnp.where(eq, sub64, NCAND), axis=0, keepdims=True)
        pick = sub64 == first
        mask = mask | pick
        rem = jnp.where(pick, -1.0, rem)

    w = jnp.where(mask, jnp.exp(score), 0.0)
    den = jnp.sum(w, axis=0, keepdims=True) + _E1

    for c in range(D // 4):
        cer4 = dot(rcat_ref[c * 256:(c + 1) * 256, :], hselt)
        for dd in range(4):
            d = 4 * c + dd
            num_d = jnp.sum(w * cer4[dd * NCAND:(dd + 1) * NCAND, :],
                            axis=0, keepdims=True)
            out_ref[d:d + 1, :] = (num_d + _E1 * xct[d:d + 1, :]) / den


def _main(x_emb_pad, tsl_col, tcat, rcat,
          w1s, bs1, ws2, bs2, woa, wob, bo1, wo2, bo2, te):
    def xspec(k):
        return pl.BlockSpec(
            (USER, PADW),
            lambda s, k=k: (jnp.where(s >= k, s - k, s), 0))

    full = lambda shp: pl.BlockSpec(shp, lambda s: (0,) * len(shp))
    in_specs = [
        xspec(4), xspec(3), xspec(2), xspec(1), xspec(0),
        pl.BlockSpec((USER, 1), lambda s: (s, 0)),
        full((D * NCAND, NCENT)),
        full((D * NCAND, NCENT)),
        full((D, 40)), full((D, 40)), full((D, 40)), full((D, 40)),
        full((D, 40)),
        full((1, 40)),
        full((40, D)), full((1, D)),
        full((D, 40)), full((D, 40)), full((1, 40)),
        full((40, D)), full((1, D)),
        full((4, D)),
    ]
    return pl.pallas_call(
        _main_body,
        grid=(SEQ,),
        in_specs=in_specs,
        out_specs=pl.BlockSpec((D, USER), lambda s: (0, s)),
        out_shape=jax.ShapeDtypeStruct((D, T), jnp.float32),
    )(x_emb_pad, x_emb_pad, x_emb_pad, x_emb_pad, x_emb_pad, tsl_col,
      tcat, rcat,
      w1s[0], w1s[1], w1s[2], w1s[3], w1s[4], bs1, ws2, bs2,
      woa, wob, bo1, wo2, bo2, te)


def kernel(x, t_slot, y, y_t_slot, vecs_use, I_array, cand_table,
           time_embeddings, Ws1, bs1, Ws2, bs2, Wo1, bo1, Wo2, bo2,
           Wi1, bi1, Wi2, bi2):
    del y, y_t_slot
    xv = x.reshape(-1).astype(jnp.int32)
    tsl_col = t_slot.reshape(-1, 1).astype(jnp.int32)
    cand_flat = cand_table.reshape(-1).astype(jnp.int32)

    vpad = jnp.pad(
        jnp.concatenate(
            [vecs_use, I_array.astype(jnp.float32)[:, None]], axis=1),
        ((0, 0), (0, PADW - D - 1)))

    x_emb_pad, candr_pad = _sc_gather(vpad, xv, cand_flat)
    candr = candr_pad[:, :D]

    te2 = time_embeddings[2:3]
    tabt = _prep(candr, te2, Wi1[:D], Wi1[D:], bi1.reshape(1, -1),
                 Wi2, bi2.reshape(1, -1))

    tcat = tabt.reshape(NCENT, NCAND, D).transpose(2, 1, 0).reshape(
        D * NCAND, NCENT)
    rcat = candr.reshape(NCENT, NCAND, D).transpose(2, 1, 0).reshape(
        D * NCAND, NCENT)

    w1s = [Ws1[0:D], Ws1[D:2 * D], Ws1[2 * D:3 * D], Ws1[3 * D:4 * D],
           Ws1[4 * D:5 * D]]
    out_t = _main(x_emb_pad, tsl_col, tcat, rcat,
                  w1s, bs1.reshape(1, -1), Ws2, bs2.reshape(1, -1),
                  Wo1[:D], Wo1[D:], bo1.reshape(1, -1), Wo2,
                  bo2.reshape(1, -1), time_embeddings)
    return out_t.T
